# Initial kernel scaffold; baseline (speedup 1.0000x reference)
import jax
import jax.numpy as jnp
from jax.experimental import pallas as pl


def kernel(x, edge_index, W1, b1, W2, b2):
    raise NotImplementedError("write your pallas kernel here")


# speedup vs baseline: 11.9159x; 11.9159x over previous
import functools

import jax
import jax.numpy as jnp
from jax import lax
from jax.experimental import pallas as pl
from jax.experimental.pallas import tpu as pltpu
from jax.experimental.pallas import tpu_sc as plsc

N = 10000
E = 320000
DF = 128
DC = 64

NC = 2
NS = 16
NW = NC * NS
EPT = E // NW
CH = 80
NCH = EPT // CH
RSTRIDE = 624
JR = 80
NJ = 8

_MESH = dict(core_axis_name="c", subcore_axis_name="s")


def _make_agg(D):
    @functools.partial(
        pl.kernel,
        mesh=plsc.VectorSubcoreMesh(**_MESH),
        out_type=jax.ShapeDtypeStruct((NC * N, D), jnp.float32),
        scratch_types=[
            pltpu.VMEM((CH,), jnp.int32),
            pltpu.VMEM((CH,), jnp.int32),
            pltpu.VMEM((CH, D), jnp.float32),
            pltpu.VMEM((JR, D), jnp.float32),
            pltpu.VMEM_SHARED((N, D), jnp.float32),
            pltpu.SemaphoreType.DMA,
        ],
    )
    def agg(z_hbm, src_hbm, dst_hbm, out_hbm, src_v, dst_v, rows_v, jbuf,
            acc, sem):
        c = lax.axis_index("c")
        s = lax.axis_index("s")
        wid = s * NC + c
        row0 = s * RSTRIDE
        for j in range(NJ):
            r = row0 + j * JR
            pltpu.sync_copy(z_hbm.at[pl.ds(r, JR)], jbuf)
            pltpu.sync_copy(jbuf, acc.at[pl.ds(r, JR)])
        plsc.subcore_barrier()
        base = wid * EPT

        def body(g, carry):
            off = base + g * CH
            pltpu.sync_copy(src_hbm.at[pl.ds(off, CH)], src_v)
            pltpu.sync_copy(dst_hbm.at[pl.ds(off, CH)], dst_v)
            pltpu.async_copy(z_hbm.at[src_v], rows_v, sem).wait()
            pltpu.sync_copy(rows_v, acc.at[dst_v], add=True)
            return carry

        lax.fori_loop(0, NCH, body, 0)
        plsc.subcore_barrier()
        for j in range(NJ):
            r = row0 + j * JR
            pltpu.sync_copy(acc.at[pl.ds(r, JR)], jbuf)
            pltpu.sync_copy(jbuf, out_hbm.at[pl.ds(c * N + r, JR)])

    return agg


_agg128 = _make_agg(DF)


EB = 8000
NHI = 80
NLO = 128


def _hist_body(dcol_ref, drow_ref, o_ref):
    i = pl.program_id(0)

    @pl.when(i == 0)
    def _init():
        o_ref[...] = jnp.zeros_like(o_ref)

    dcol = dcol_ref[...]
    drow = drow_ref[0]
    hi_t = (lax.broadcasted_iota(jnp.int32, (NHI, EB), 0)
            == (drow >> 7)).astype(jnp.float32)
    lo = (lax.broadcasted_iota(jnp.int32, (EB, NLO), 1)
          == (dcol & 127)).astype(jnp.float32)
    o_ref[...] += jnp.dot(hi_t, lo, preferred_element_type=jnp.float32)


def _hist(dcol, drow):
    return pl.pallas_call(
        _hist_body,
        grid=(E // EB,),
        in_specs=[
            pl.BlockSpec((EB, 1), lambda i: (i, 0)),
            pl.BlockSpec((1, 1, EB), lambda i: (i, 0, 0)),
        ],
        out_specs=pl.BlockSpec((NHI, NLO), lambda i: (0, 0)),
        out_shape=jax.ShapeDtypeStruct((NHI, NLO), jnp.float32),
    )(dcol, drow)


BR = 1000


def _dinv_of(deg_ref):
    return lax.rsqrt(deg_ref[...] + 1.0)


def _mm_scale_body(x_ref, w_ref, deg_ref, o_ref):
    dinv = _dinv_of(deg_ref)
    xw = jnp.dot(x_ref[...], w_ref[...], preferred_element_type=jnp.float32)
    o_ref[...] = xw * dinv


def _mm_scale(x, W1, degc):
    return pl.pallas_call(
        _mm_scale_body,
        grid=(N // BR,),
        in_specs=[
            pl.BlockSpec((BR, DF), lambda i: (i, 0)),
            pl.BlockSpec((DF, DF), lambda i: (0, 0)),
            pl.BlockSpec((BR, 1), lambda i: (i, 0)),
        ],
        out_specs=pl.BlockSpec((BR, DF), lambda i: (i, 0)),
        out_shape=jax.ShapeDtypeStruct((N, DF), jnp.float32),
    )(x, W1, degc)


def _layer2_body(p_ref, z_ref, deg_ref, b1_ref, o_ref):
    dinv = _dinv_of(deg_ref)
    agg = (p_ref[0] + p_ref[1] - z_ref[...]) * dinv + b1_ref[...]
    o_ref[...] = jnp.maximum(agg, 0.0) * dinv


def _layer2(p1, z, degc, b1r):
    return pl.pallas_call(
        _layer2_body,
        grid=(N // BR,),
        in_specs=[
            pl.BlockSpec((NC, BR, DF), lambda i: (0, i, 0)),
            pl.BlockSpec((BR, DF), lambda i: (i, 0)),
            pl.BlockSpec((BR, 1), lambda i: (i, 0)),
            pl.BlockSpec((1, DF), lambda i: (0, 0)),
        ],
        out_specs=pl.BlockSpec((BR, DF), lambda i: (i, 0)),
        out_shape=jax.ShapeDtypeStruct((N, DF), jnp.float32),
    )(p1, z, degc, b1r)


def _final_body(p_ref, z2_ref, deg_ref, w2_ref, b2_ref, o_ref):
    dinv = _dinv_of(deg_ref)
    agg = (p_ref[0] + p_ref[1] - z2_ref[...]) * dinv
    v = jnp.dot(agg, w2_ref[...], preferred_element_type=jnp.float32)
    v = v + b2_ref[...]
    col = lax.broadcasted_iota(jnp.int32, (BR, DC), 1)
    valid = col < 40
    vm = jnp.where(valid, v, jnp.float32(-1e30))
    m = jnp.max(vm, axis=1, keepdims=True)
    ex = jnp.where(valid, jnp.exp(v - m), 0.0)
    lse = jnp.log(jnp.sum(ex, axis=1, keepdims=True))
    o_ref[...] = v - m - lse


def _final(p2, z2, degc, W2p, b2r):
    return pl.pallas_call(
        _final_body,
        grid=(N // BR,),
        in_specs=[
            pl.BlockSpec((NC, BR, DF), lambda i: (0, i, 0)),
            pl.BlockSpec((BR, DF), lambda i: (i, 0)),
            pl.BlockSpec((BR, 1), lambda i: (i, 0)),
            pl.BlockSpec((DF, DC), lambda i: (0, 0)),
            pl.BlockSpec((1, DC), lambda i: (0, 0)),
        ],
        out_specs=pl.BlockSpec((BR, DC), lambda i: (i, 0)),
        out_shape=jax.ShapeDtypeStruct((N, DC), jnp.float32),
    )(p2, z2, degc, W2p, b2r)


def kernel(x, edge_index, W1, b1, W2, b2):
    ei = edge_index.astype(jnp.int32)
    src = ei[0]
    dst = ei[1]
    deg2d = _hist(dst.reshape(E, 1), dst.reshape(E // EB, 1, EB))
    degc = deg2d.reshape(NHI * NLO)[:N].reshape(N, 1)
    z = _mm_scale(x, W1, degc)
    p1 = _agg128(z, src, dst).reshape(NC, N, DF)
    z2 = _layer2(p1, z, degc, b1.reshape(1, DF))
    p2 = _agg128(z2, src, dst).reshape(NC, N, DF)
    W2p = jnp.pad(W2, ((0, 0), (0, DC - W2.shape[1])))
    b2r = jnp.pad(b2, (0, DC - b2.shape[0])).reshape(1, DC)
    out64 = _final(p2, z2, degc, W2p, b2r)
    return out64[:, :40]


# speedup vs baseline: 19.2954x; 1.6193x over previous
import functools

import jax
import jax.numpy as jnp
from jax import lax
from jax.experimental import pallas as pl
from jax.experimental.pallas import tpu as pltpu
from jax.experimental.pallas import tpu_sc as plsc

N = 10000
E = 320000
DF = 128
DC = 64

NC = 2
NS = 16
NW = NC * NS
EPT = E // NW
CH = 40
NCH = EPT // CH
U = 5
NWIN = NCH // U
RSTRIDE = 624
JR = 40
NJ = 16

_MESH = dict(core_axis_name="c", subcore_axis_name="s")


def _make_agg(D):
    scratch = (
        [pltpu.VMEM((CH,), jnp.int32) for _ in range(U)]
        + [pltpu.VMEM((CH,), jnp.int32) for _ in range(U)]
        + [pltpu.VMEM((CH, D), jnp.float32) for _ in range(U)]
        + [pltpu.VMEM((JR, D), jnp.float32)]
        + [pltpu.VMEM_SHARED((N, D), jnp.float32)]
        + [pltpu.SemaphoreType.DMA for _ in range(2 * U)]
    )

    @functools.partial(
        pl.kernel,
        mesh=plsc.VectorSubcoreMesh(**_MESH),
        out_type=jax.ShapeDtypeStruct((NC * N, D), jnp.float32),
        scratch_types=scratch,
    )
    def agg(z_hbm, src_hbm, dst_hbm, out_hbm, *sc):
        src_v = sc[0:U]
        dst_v = sc[U:2 * U]
        rows_v = sc[2 * U:3 * U]
        jbuf = sc[3 * U]
        acc = sc[3 * U + 1]
        isem = sc[3 * U + 2:3 * U + 2 + U]
        gsem = sc[3 * U + 2 + U:3 * U + 2 + 2 * U]
        c = lax.axis_index("c")
        s = lax.axis_index("s")
        wid = s * NC + c
        row0 = s * RSTRIDE
        for j in range(NJ):
            r = row0 + j * JR
            pltpu.sync_copy(z_hbm.at[pl.ds(r, JR)], jbuf)
            pltpu.sync_copy(jbuf, acc.at[pl.ds(r, JR)])
        base = wid * EPT
        for b in range(U):
            off = base + b * CH
            pltpu.async_copy(src_hbm.at[pl.ds(off, CH)], src_v[b], isem[b])
            pltpu.async_copy(dst_hbm.at[pl.ds(off, CH)], dst_v[b], isem[b])
        plsc.subcore_barrier()

        def window(gg, carry):
            hg = []
            for b in range(U):
                pltpu.make_async_copy(src_hbm.at[pl.ds(base, CH)],
                                      src_v[b], isem[b]).wait()
                pltpu.make_async_copy(dst_hbm.at[pl.ds(base, CH)],
                                      dst_v[b], isem[b]).wait()
                hg.append(pltpu.async_copy(z_hbm.at[src_v[b]], rows_v[b],
                                           gsem[b]))
            hs = []
            for b in range(U):
                hg[b].wait()
                hs.append(pltpu.async_copy(rows_v[b], acc.at[dst_v[b]],
                                           gsem[b], add=True))
            nxt = base + (gg + 1) * U * CH
            for b in range(U):
                hs[b].wait()

                @pl.when(gg < NWIN - 1)
                def _prefetch(b=b):
                    off = nxt + b * CH
                    pltpu.async_copy(src_hbm.at[pl.ds(off, CH)], src_v[b],
                                     isem[b])
                    pltpu.async_copy(dst_hbm.at[pl.ds(off, CH)], dst_v[b],
                                     isem[b])
            return carry

        lax.fori_loop(0, NWIN, window, 0)
        plsc.subcore_barrier()
        for j in range(NJ):
            r = row0 + j * JR
            pltpu.sync_copy(acc.at[pl.ds(r, JR)], jbuf)
            pltpu.sync_copy(jbuf, out_hbm.at[pl.ds(c * N + r, JR)])

    return agg


_agg128 = _make_agg(DF)


EB = 8000
NHI = 80
NLO = 128


def _hist_body(dcol_ref, drow_ref, o_ref):
    i = pl.program_id(0)

    @pl.when(i == 0)
    def _init():
        o_ref[...] = jnp.zeros_like(o_ref)

    dcol = dcol_ref[...]
    drow = drow_ref[0]
    hi_t = (lax.broadcasted_iota(jnp.int32, (NHI, EB), 0)
            == (drow >> 7)).astype(jnp.float32)
    lo = (lax.broadcasted_iota(jnp.int32, (EB, NLO), 1)
          == (dcol & 127)).astype(jnp.float32)
    o_ref[...] += jnp.dot(hi_t, lo, preferred_element_type=jnp.float32)


def _hist(dcol, drow):
    return pl.pallas_call(
        _hist_body,
        grid=(E // EB,),
        in_specs=[
            pl.BlockSpec((EB, 1), lambda i: (i, 0)),
            pl.BlockSpec((1, 1, EB), lambda i: (i, 0, 0)),
        ],
        out_specs=pl.BlockSpec((NHI, NLO), lambda i: (0, 0)),
        out_shape=jax.ShapeDtypeStruct((NHI, NLO), jnp.float32),
    )(dcol, drow)


BR = 1000


def _dinv_of(deg_ref):
    return lax.rsqrt(deg_ref[...] + 1.0)


def _mm_scale_body(x_ref, w_ref, deg_ref, o_ref):
    dinv = _dinv_of(deg_ref)
    xw = jnp.dot(x_ref[...], w_ref[...], preferred_element_type=jnp.float32)
    o_ref[...] = xw * dinv


def _mm_scale(x, W1, degc):
    return pl.pallas_call(
        _mm_scale_body,
        grid=(N // BR,),
        in_specs=[
            pl.BlockSpec((BR, DF), lambda i: (i, 0)),
            pl.BlockSpec((DF, DF), lambda i: (0, 0)),
            pl.BlockSpec((BR, 1), lambda i: (i, 0)),
        ],
        out_specs=pl.BlockSpec((BR, DF), lambda i: (i, 0)),
        out_shape=jax.ShapeDtypeStruct((N, DF), jnp.float32),
    )(x, W1, degc)


def _layer2_body(p_ref, z_ref, deg_ref, b1_ref, o_ref):
    dinv = _dinv_of(deg_ref)
    agg = (p_ref[0] + p_ref[1] - z_ref[...]) * dinv + b1_ref[...]
    o_ref[...] = jnp.maximum(agg, 0.0) * dinv


def _layer2(p1, z, degc, b1r):
    return pl.pallas_call(
        _layer2_body,
        grid=(N // BR,),
        in_specs=[
            pl.BlockSpec((NC, BR, DF), lambda i: (0, i, 0)),
            pl.BlockSpec((BR, DF), lambda i: (i, 0)),
            pl.BlockSpec((BR, 1), lambda i: (i, 0)),
            pl.BlockSpec((1, DF), lambda i: (0, 0)),
        ],
        out_specs=pl.BlockSpec((BR, DF), lambda i: (i, 0)),
        out_shape=jax.ShapeDtypeStruct((N, DF), jnp.float32),
    )(p1, z, degc, b1r)


def _final_body(p_ref, z2_ref, deg_ref, w2_ref, b2_ref, o_ref):
    dinv = _dinv_of(deg_ref)
    agg = (p_ref[0] + p_ref[1] - z2_ref[...]) * dinv
    v = jnp.dot(agg, w2_ref[...], preferred_element_type=jnp.float32)
    v = v + b2_ref[...]
    col = lax.broadcasted_iota(jnp.int32, (BR, DC), 1)
    valid = col < 40
    vm = jnp.where(valid, v, jnp.float32(-1e30))
    m = jnp.max(vm, axis=1, keepdims=True)
    ex = jnp.where(valid, jnp.exp(v - m), 0.0)
    lse = jnp.log(jnp.sum(ex, axis=1, keepdims=True))
    o_ref[...] = v - m - lse


def _final(p2, z2, degc, W2p, b2r):
    return pl.pallas_call(
        _final_body,
        grid=(N // BR,),
        in_specs=[
            pl.BlockSpec((NC, BR, DF), lambda i: (0, i, 0)),
            pl.BlockSpec((BR, DF), lambda i: (i, 0)),
            pl.BlockSpec((BR, 1), lambda i: (i, 0)),
            pl.BlockSpec((DF, DC), lambda i: (0, 0)),
            pl.BlockSpec((1, DC), lambda i: (0, 0)),
        ],
        out_specs=pl.BlockSpec((BR, DC), lambda i: (i, 0)),
        out_shape=jax.ShapeDtypeStruct((N, DC), jnp.float32),
    )(p2, z2, degc, W2p, b2r)


def kernel(x, edge_index, W1, b1, W2, b2):
    ei = edge_index.astype(jnp.int32)
    src = ei[0]
    dst = ei[1]
    deg2d = _hist(dst.reshape(E, 1), dst.reshape(E // EB, 1, EB))
    degc = deg2d.reshape(NHI * NLO)[:N].reshape(N, 1)
    z = _mm_scale(x, W1, degc)
    p1 = _agg128(z, src, dst).reshape(NC, N, DF)
    z2 = _layer2(p1, z, degc, b1.reshape(1, DF))
    p2 = _agg128(z2, src, dst).reshape(NC, N, DF)
    W2p = jnp.pad(W2, ((0, 0), (0, DC - W2.shape[1])))
    b2r = jnp.pad(b2, (0, DC - b2.shape[0])).reshape(1, DC)
    out64 = _final(p2, z2, degc, W2p, b2r)
    return out64[:, :40]


# speedup vs baseline: 20.3198x; 1.0531x over previous
import functools

import jax
import jax.numpy as jnp
from jax import lax
from jax.experimental import pallas as pl
from jax.experimental.pallas import tpu as pltpu
from jax.experimental.pallas import tpu_sc as plsc

N = 10000
E = 320000
DF = 128
DC = 64

NC = 2
NS = 16
NW = NC * NS
EPT = E // NW
CH = 40
NCH = EPT // CH
U = 5
NWIN = NCH // U
RSTRIDE = 624
JR = 40
NJ = 16

_MESH = dict(core_axis_name="c", subcore_axis_name="s")


def _make_agg(D):
    scratch = (
        [pltpu.VMEM((CH,), jnp.int32) for _ in range(U)]
        + [pltpu.VMEM((CH,), jnp.int32) for _ in range(U)]
        + [pltpu.VMEM((CH, D), jnp.float32) for _ in range(U)]
        + [pltpu.VMEM_SHARED((N, D), jnp.float32)]
        + [pltpu.SemaphoreType.DMA for _ in range(2 * U)]
    )

    @functools.partial(
        pl.kernel,
        mesh=plsc.VectorSubcoreMesh(**_MESH),
        out_type=jax.ShapeDtypeStruct((NC * N, D), jnp.float32),
        scratch_types=scratch,
    )
    def agg(z_hbm, src_hbm, dst_hbm, out_hbm, *sc):
        src_v = sc[0:U]
        dst_v = sc[U:2 * U]
        rows_v = sc[2 * U:3 * U]
        acc = sc[3 * U]
        isem = sc[3 * U + 1:3 * U + 1 + U]
        gsem = sc[3 * U + 1 + U:3 * U + 1 + 2 * U]
        c = lax.axis_index("c")
        s = lax.axis_index("s")
        wid = s * NC + c
        row0 = s * RSTRIDE
        pltpu.sync_copy(z_hbm.at[pl.ds(row0, NJ * JR)],
                        acc.at[pl.ds(row0, NJ * JR)])
        base = wid * EPT
        for b in range(U):
            off = base + b * CH
            pltpu.async_copy(src_hbm.at[pl.ds(off, CH)], src_v[b], isem[b])
            pltpu.async_copy(dst_hbm.at[pl.ds(off, CH)], dst_v[b], isem[b])
        plsc.subcore_barrier()

        def window(gg, carry):
            hg = []
            for b in range(U):
                pltpu.make_async_copy(src_hbm.at[pl.ds(base, CH)],
                                      src_v[b], isem[b]).wait()
                pltpu.make_async_copy(dst_hbm.at[pl.ds(base, CH)],
                                      dst_v[b], isem[b]).wait()
                hg.append(pltpu.async_copy(z_hbm.at[src_v[b]], rows_v[b],
                                           gsem[b]))
            hs = []
            for b in range(U):
                hg[b].wait()
                hs.append(pltpu.async_copy(rows_v[b], acc.at[dst_v[b]],
                                           gsem[b], add=True))
            nxt = base + (gg + 1) * U * CH
            for b in range(U):
                hs[b].wait()

                @pl.when(gg < NWIN - 1)
                def _prefetch(b=b):
                    off = nxt + b * CH
                    pltpu.async_copy(src_hbm.at[pl.ds(off, CH)], src_v[b],
                                     isem[b])
                    pltpu.async_copy(dst_hbm.at[pl.ds(off, CH)], dst_v[b],
                                     isem[b])
            return carry

        lax.fori_loop(0, NWIN, window, 0)
        plsc.subcore_barrier()
        pltpu.sync_copy(acc.at[pl.ds(row0, NJ * JR)],
                        out_hbm.at[pl.ds(c * N + row0, NJ * JR)])

    return agg


_agg128 = _make_agg(DF)


EB = 8000
NHI = 80
NLO = 128


def _hist_body(dcol_ref, drow_ref, o_ref):
    i = pl.program_id(0)

    @pl.when(i == 0)
    def _init():
        o_ref[...] = jnp.zeros_like(o_ref)

    dcol = dcol_ref[...]
    drow = drow_ref[0]
    hi_t = (lax.broadcasted_iota(jnp.int32, (NHI, EB), 0)
            == (drow >> 7)).astype(jnp.float32)
    lo = (lax.broadcasted_iota(jnp.int32, (EB, NLO), 1)
          == (dcol & 127)).astype(jnp.float32)
    o_ref[...] += jnp.dot(hi_t, lo, preferred_element_type=jnp.float32)


def _hist(dcol, drow):
    return pl.pallas_call(
        _hist_body,
        grid=(E // EB,),
        in_specs=[
            pl.BlockSpec((EB, 1), lambda i: (i, 0)),
            pl.BlockSpec((1, 1, EB), lambda i: (i, 0, 0)),
        ],
        out_specs=pl.BlockSpec((NHI, NLO), lambda i: (0, 0)),
        out_shape=jax.ShapeDtypeStruct((NHI, NLO), jnp.float32),
    )(dcol, drow)


BR = 1000


def _dinv_of(deg_ref):
    return lax.rsqrt(deg_ref[...] + 1.0)


def _mm_scale_body(x_ref, w_ref, deg_ref, o_ref):
    dinv = _dinv_of(deg_ref)
    xw = jnp.dot(x_ref[...], w_ref[...], preferred_element_type=jnp.float32)
    o_ref[...] = xw * dinv


def _mm_scale(x, W1, degc):
    return pl.pallas_call(
        _mm_scale_body,
        grid=(N // BR,),
        in_specs=[
            pl.BlockSpec((BR, DF), lambda i: (i, 0)),
            pl.BlockSpec((DF, DF), lambda i: (0, 0)),
            pl.BlockSpec((BR, 1), lambda i: (i, 0)),
        ],
        out_specs=pl.BlockSpec((BR, DF), lambda i: (i, 0)),
        out_shape=jax.ShapeDtypeStruct((N, DF), jnp.float32),
    )(x, W1, degc)


def _layer2_body(p_ref, z_ref, deg_ref, b1_ref, o_ref):
    dinv = _dinv_of(deg_ref)
    agg = (p_ref[0] + p_ref[1] - z_ref[...]) * dinv + b1_ref[...]
    o_ref[...] = jnp.maximum(agg, 0.0) * dinv


def _layer2(p1, z, degc, b1r):
    return pl.pallas_call(
        _layer2_body,
        grid=(N // BR,),
        in_specs=[
            pl.BlockSpec((NC, BR, DF), lambda i: (0, i, 0)),
            pl.BlockSpec((BR, DF), lambda i: (i, 0)),
            pl.BlockSpec((BR, 1), lambda i: (i, 0)),
            pl.BlockSpec((1, DF), lambda i: (0, 0)),
        ],
        out_specs=pl.BlockSpec((BR, DF), lambda i: (i, 0)),
        out_shape=jax.ShapeDtypeStruct((N, DF), jnp.float32),
    )(p1, z, degc, b1r)


def _final_body(p_ref, z2_ref, deg_ref, w2_ref, b2_ref, o_ref):
    dinv = _dinv_of(deg_ref)
    agg = (p_ref[0] + p_ref[1] - z2_ref[...]) * dinv
    v = jnp.dot(agg, w2_ref[...], preferred_element_type=jnp.float32)
    v = v + b2_ref[...]
    col = lax.broadcasted_iota(jnp.int32, (BR, DC), 1)
    valid = col < 40
    vm = jnp.where(valid, v, jnp.float32(-1e30))
    m = jnp.max(vm, axis=1, keepdims=True)
    ex = jnp.where(valid, jnp.exp(v - m), 0.0)
    lse = jnp.log(jnp.sum(ex, axis=1, keepdims=True))
    o_ref[...] = v - m - lse


def _final(p2, z2, degc, W2p, b2r):
    return pl.pallas_call(
        _final_body,
        grid=(N // BR,),
        in_specs=[
            pl.BlockSpec((NC, BR, DF), lambda i: (0, i, 0)),
            pl.BlockSpec((BR, DF), lambda i: (i, 0)),
            pl.BlockSpec((BR, 1), lambda i: (i, 0)),
            pl.BlockSpec((DF, DC), lambda i: (0, 0)),
            pl.BlockSpec((1, DC), lambda i: (0, 0)),
        ],
        out_specs=pl.BlockSpec((BR, DC), lambda i: (i, 0)),
        out_shape=jax.ShapeDtypeStruct((N, DC), jnp.float32),
    )(p2, z2, degc, W2p, b2r)


def kernel(x, edge_index, W1, b1, W2, b2):
    ei = edge_index.astype(jnp.int32)
    src = ei[0]
    dst = ei[1]
    deg2d = _hist(dst.reshape(E, 1), dst.reshape(E // EB, 1, EB))
    degc = deg2d.reshape(NHI * NLO)[:N].reshape(N, 1)
    z = _mm_scale(x, W1, degc)
    p1 = _agg128(z, src, dst).reshape(NC, N, DF)
    z2 = _layer2(p1, z, degc, b1.reshape(1, DF))
    p2 = _agg128(z2, src, dst).reshape(NC, N, DF)
    W2p = jnp.pad(W2, ((0, 0), (0, DC - W2.shape[1])))
    b2r = jnp.pad(b2, (0, DC - b2.shape[0])).reshape(1, DC)
    out64 = _final(p2, z2, degc, W2p, b2r)
    return out64[:, :40]


# speedup vs baseline: 20.4024x; 1.0041x over previous
import functools

import jax
import jax.numpy as jnp
from jax import lax
from jax.experimental import pallas as pl
from jax.experimental.pallas import tpu as pltpu
from jax.experimental.pallas import tpu_sc as plsc

N = 10000
E = 320000
DF = 128
DC = 64

NC = 2
NS = 16
NW = NC * NS
EPT = E // NW
CH = 80
NCH = EPT // CH
U = 4
NWIN = NCH // U
TAIL = NCH - NWIN * U
RSTRIDE = 624
JR = 40
NJ = 16

_MESH = dict(core_axis_name="c", subcore_axis_name="s")


def _make_agg(D):
    scratch = (
        [pltpu.VMEM((CH,), jnp.int32) for _ in range(U)]
        + [pltpu.VMEM((CH,), jnp.int32) for _ in range(U)]
        + [pltpu.VMEM((CH, D), jnp.float32) for _ in range(U)]
        + [pltpu.VMEM_SHARED((N, D), jnp.float32)]
        + [pltpu.SemaphoreType.DMA for _ in range(2 * U)]
    )

    @functools.partial(
        pl.kernel,
        mesh=plsc.VectorSubcoreMesh(**_MESH),
        out_type=jax.ShapeDtypeStruct((NC * N, D), jnp.float32),
        scratch_types=scratch,
    )
    def agg(z_hbm, src_hbm, dst_hbm, out_hbm, *sc):
        src_v = sc[0:U]
        dst_v = sc[U:2 * U]
        rows_v = sc[2 * U:3 * U]
        acc = sc[3 * U]
        isem = sc[3 * U + 1:3 * U + 1 + U]
        gsem = sc[3 * U + 1 + U:3 * U + 1 + 2 * U]
        c = lax.axis_index("c")
        s = lax.axis_index("s")
        wid = s * NC + c
        row0 = s * RSTRIDE
        pltpu.sync_copy(z_hbm.at[pl.ds(row0, NJ * JR)],
                        acc.at[pl.ds(row0, NJ * JR)])
        base = wid * EPT
        for b in range(U):
            off = base + b * CH
            pltpu.async_copy(src_hbm.at[pl.ds(off, CH)], src_v[b], isem[b])
            pltpu.async_copy(dst_hbm.at[pl.ds(off, CH)], dst_v[b], isem[b])
        plsc.subcore_barrier()

        def window(gg, carry):
            hg = []
            for b in range(U):
                pltpu.make_async_copy(src_hbm.at[pl.ds(base, CH)],
                                      src_v[b], isem[b]).wait()
                pltpu.make_async_copy(dst_hbm.at[pl.ds(base, CH)],
                                      dst_v[b], isem[b]).wait()
                hg.append(pltpu.async_copy(z_hbm.at[src_v[b]], rows_v[b],
                                           gsem[b]))
            hs = []
            for b in range(U):
                hg[b].wait()
                hs.append(pltpu.async_copy(rows_v[b], acc.at[dst_v[b]],
                                           gsem[b], add=True))
            nxt = base + (gg + 1) * U * CH
            for b in range(U):
                hs[b].wait()

                @pl.when(gg < NWIN - 1)
                def _prefetch(b=b):
                    off = nxt + b * CH
                    pltpu.async_copy(src_hbm.at[pl.ds(off, CH)], src_v[b],
                                     isem[b])
                    pltpu.async_copy(dst_hbm.at[pl.ds(off, CH)], dst_v[b],
                                     isem[b])
            return carry

        lax.fori_loop(0, NWIN, window, 0)
        for t in range(TAIL):
            off = base + (NWIN * U + t) * CH
            pltpu.sync_copy(src_hbm.at[pl.ds(off, CH)], src_v[0])
            pltpu.sync_copy(dst_hbm.at[pl.ds(off, CH)], dst_v[0])
            pltpu.async_copy(z_hbm.at[src_v[0]], rows_v[0], gsem[0]).wait()
            pltpu.sync_copy(rows_v[0], acc.at[dst_v[0]], add=True)
        plsc.subcore_barrier()
        pltpu.sync_copy(acc.at[pl.ds(row0, NJ * JR)],
                        out_hbm.at[pl.ds(c * N + row0, NJ * JR)])

    return agg


_agg128 = _make_agg(DF)


EB = 8000
NHI = 80
NLO = 128


def _hist_body(dcol_ref, drow_ref, o_ref):
    i = pl.program_id(0)

    @pl.when(i == 0)
    def _init():
        o_ref[...] = jnp.zeros_like(o_ref)

    dcol = dcol_ref[...]
    drow = drow_ref[0]
    hi_t = (lax.broadcasted_iota(jnp.int32, (NHI, EB), 0)
            == (drow >> 7)).astype(jnp.float32)
    lo = (lax.broadcasted_iota(jnp.int32, (EB, NLO), 1)
          == (dcol & 127)).astype(jnp.float32)
    o_ref[...] += jnp.dot(hi_t, lo, preferred_element_type=jnp.float32)


def _hist(dcol, drow):
    return pl.pallas_call(
        _hist_body,
        grid=(E // EB,),
        in_specs=[
            pl.BlockSpec((EB, 1), lambda i: (i, 0)),
            pl.BlockSpec((1, 1, EB), lambda i: (i, 0, 0)),
        ],
        out_specs=pl.BlockSpec((NHI, NLO), lambda i: (0, 0)),
        out_shape=jax.ShapeDtypeStruct((NHI, NLO), jnp.float32),
    )(dcol, drow)


BR = 1000


def _dinv_of(deg_ref):
    return lax.rsqrt(deg_ref[...] + 1.0)


def _mm_scale_body(x_ref, w_ref, deg_ref, o_ref):
    dinv = _dinv_of(deg_ref)
    xw = jnp.dot(x_ref[...], w_ref[...], preferred_element_type=jnp.float32)
    o_ref[...] = xw * dinv


def _mm_scale(x, W1, degc):
    return pl.pallas_call(
        _mm_scale_body,
        grid=(N // BR,),
        in_specs=[
            pl.BlockSpec((BR, DF), lambda i: (i, 0)),
            pl.BlockSpec((DF, DF), lambda i: (0, 0)),
            pl.BlockSpec((BR, 1), lambda i: (i, 0)),
        ],
        out_specs=pl.BlockSpec((BR, DF), lambda i: (i, 0)),
        out_shape=jax.ShapeDtypeStruct((N, DF), jnp.float32),
    )(x, W1, degc)


def _layer2_body(p_ref, z_ref, deg_ref, b1_ref, o_ref):
    dinv = _dinv_of(deg_ref)
    agg = (p_ref[0] + p_ref[1] - z_ref[...]) * dinv + b1_ref[...]
    o_ref[...] = jnp.maximum(agg, 0.0) * dinv


def _layer2(p1, z, degc, b1r):
    return pl.pallas_call(
        _layer2_body,
        grid=(N // BR,),
        in_specs=[
            pl.BlockSpec((NC, BR, DF), lambda i: (0, i, 0)),
            pl.BlockSpec((BR, DF), lambda i: (i, 0)),
            pl.BlockSpec((BR, 1), lambda i: (i, 0)),
            pl.BlockSpec((1, DF), lambda i: (0, 0)),
        ],
        out_specs=pl.BlockSpec((BR, DF), lambda i: (i, 0)),
        out_shape=jax.ShapeDtypeStruct((N, DF), jnp.float32),
    )(p1, z, degc, b1r)


def _final_body(p_ref, z2_ref, deg_ref, w2_ref, b2_ref, o_ref):
    dinv = _dinv_of(deg_ref)
    agg = (p_ref[0] + p_ref[1] - z2_ref[...]) * dinv
    v = jnp.dot(agg, w2_ref[...], preferred_element_type=jnp.float32)
    v = v + b2_ref[...]
    col = lax.broadcasted_iota(jnp.int32, (BR, DC), 1)
    valid = col < 40
    vm = jnp.where(valid, v, jnp.float32(-1e30))
    m = jnp.max(vm, axis=1, keepdims=True)
    ex = jnp.where(valid, jnp.exp(v - m), 0.0)
    lse = jnp.log(jnp.sum(ex, axis=1, keepdims=True))
    o_ref[...] = v - m - lse


def _final(p2, z2, degc, W2p, b2r):
    return pl.pallas_call(
        _final_body,
        grid=(N // BR,),
        in_specs=[
            pl.BlockSpec((NC, BR, DF), lambda i: (0, i, 0)),
            pl.BlockSpec((BR, DF), lambda i: (i, 0)),
            pl.BlockSpec((BR, 1), lambda i: (i, 0)),
            pl.BlockSpec((DF, DC), lambda i: (0, 0)),
            pl.BlockSpec((1, DC), lambda i: (0, 0)),
        ],
        out_specs=pl.BlockSpec((BR, DC), lambda i: (i, 0)),
        out_shape=jax.ShapeDtypeStruct((N, DC), jnp.float32),
    )(p2, z2, degc, W2p, b2r)


def kernel(x, edge_index, W1, b1, W2, b2):
    ei = edge_index.astype(jnp.int32)
    src = ei[0]
    dst = ei[1]
    deg2d = _hist(dst.reshape(E, 1), dst.reshape(E // EB, 1, EB))
    degc = deg2d.reshape(NHI * NLO)[:N].reshape(N, 1)
    z = _mm_scale(x, W1, degc)
    p1 = _agg128(z, src, dst).reshape(NC, N, DF)
    z2 = _layer2(p1, z, degc, b1.reshape(1, DF))
    p2 = _agg128(z2, src, dst).reshape(NC, N, DF)
    W2p = jnp.pad(W2, ((0, 0), (0, DC - W2.shape[1])))
    b2r = jnp.pad(b2, (0, DC - b2.shape[0])).reshape(1, DC)
    out64 = _final(p2, z2, degc, W2p, b2r)
    return out64[:, :40]


# speedup vs baseline: 26.6937x; 1.3084x over previous
import functools

import jax
import jax.numpy as jnp
from jax import lax
from jax.experimental import pallas as pl
from jax.experimental.pallas import tpu as pltpu
from jax.experimental.pallas import tpu_sc as plsc

N = 10000
E = 320000
DF = 128
DC = 64

NC = 2
NS = 16
NW = NC * NS
EPT = E // NW
CH = 80
NCH = EPT // CH
U = 4
NWIN = NCH // U
TAIL = NCH - NWIN * U
RSTRIDE = 624
JR = 40
NJ = 16

_MESH = dict(core_axis_name="c", subcore_axis_name="s")


def _make_agg(D):
    scratch = (
        [pltpu.VMEM((CH,), jnp.int32) for _ in range(U)]
        + [pltpu.VMEM((CH,), jnp.int32) for _ in range(U)]
        + [pltpu.VMEM((CH, D), jnp.float32) for _ in range(U)]
        + [pltpu.VMEM_SHARED((N, D), jnp.float32)]
        + [pltpu.SemaphoreType.DMA for _ in range(2 * U)]
    )

    @functools.partial(
        pl.kernel,
        mesh=plsc.VectorSubcoreMesh(**_MESH),
        out_type=jax.ShapeDtypeStruct((NC * N, D), jnp.float32),
        scratch_types=scratch,
    )
    def agg(z_hbm, src_hbm, dst_hbm, out_hbm, *sc):
        src_v = sc[0:U]
        dst_v = sc[U:2 * U]
        rows_v = sc[2 * U:3 * U]
        acc = sc[3 * U]
        isem = sc[3 * U + 1:3 * U + 1 + U]
        gsem = sc[3 * U + 1 + U:3 * U + 1 + 2 * U]
        c = lax.axis_index("c")
        s = lax.axis_index("s")
        wid = s * NC + c
        row0 = s * RSTRIDE
        pltpu.sync_copy(z_hbm.at[pl.ds(row0, NJ * JR)],
                        acc.at[pl.ds(row0, NJ * JR)])
        base = wid * EPT
        for b in range(U):
            off = base + b * CH
            pltpu.async_copy(src_hbm.at[pl.ds(off, CH)], src_v[b], isem[b])
            pltpu.async_copy(dst_hbm.at[pl.ds(off, CH)], dst_v[b], isem[b])
        plsc.subcore_barrier()

        def window(gg, carry):
            hg = []
            for b in range(U):
                pltpu.make_async_copy(src_hbm.at[pl.ds(base, CH)],
                                      src_v[b], isem[b]).wait()
                pltpu.make_async_copy(dst_hbm.at[pl.ds(base, CH)],
                                      dst_v[b], isem[b]).wait()
                hg.append(pltpu.async_copy(z_hbm.at[src_v[b]], rows_v[b],
                                           gsem[b]))
            hs = []
            for b in range(U):
                hg[b].wait()
                hs.append(pltpu.async_copy(rows_v[b], acc.at[dst_v[b]],
                                           gsem[b], add=True))
            nxt = base + (gg + 1) * U * CH
            for b in range(U):
                hs[b].wait()

                @pl.when(gg < NWIN - 1)
                def _prefetch(b=b):
                    off = nxt + b * CH
                    pltpu.async_copy(src_hbm.at[pl.ds(off, CH)], src_v[b],
                                     isem[b])
                    pltpu.async_copy(dst_hbm.at[pl.ds(off, CH)], dst_v[b],
                                     isem[b])
            return carry

        lax.fori_loop(0, NWIN, window, 0)
        for t in range(TAIL):
            off = base + (NWIN * U + t) * CH
            pltpu.sync_copy(src_hbm.at[pl.ds(off, CH)], src_v[0])
            pltpu.sync_copy(dst_hbm.at[pl.ds(off, CH)], dst_v[0])
            pltpu.async_copy(z_hbm.at[src_v[0]], rows_v[0], gsem[0]).wait()
            pltpu.sync_copy(rows_v[0], acc.at[dst_v[0]], add=True)
        plsc.subcore_barrier()
        pltpu.sync_copy(acc.at[pl.ds(row0, NJ * JR)],
                        out_hbm.at[pl.ds(c * N + row0, NJ * JR)])

    return agg


_agg128 = _make_agg(DF)


EB = 16000
NHI = 80
NLO = 128


def _hist_body(ei_ref, o_ref):
    i = pl.program_id(0)

    @pl.when(i == 0)
    def _init():
        o_ref[...] = jnp.zeros_like(o_ref)

    drow = ei_ref[1:2, :]
    hi_t = (lax.broadcasted_iota(jnp.int32, (NHI, EB), 0)
            == (drow >> 7)).astype(jnp.float32)
    lo_t = (lax.broadcasted_iota(jnp.int32, (NLO, EB), 0)
            == (drow & 127)).astype(jnp.float32)
    o_ref[...] += lax.dot_general(hi_t, lo_t, (((1,), (1,)), ((), ())),
                                  preferred_element_type=jnp.float32)


def _hist(ei):
    return pl.pallas_call(
        _hist_body,
        grid=(E // EB,),
        in_specs=[pl.BlockSpec((2, EB), lambda i: (0, i))],
        out_specs=pl.BlockSpec((NHI, NLO), lambda i: (0, 0)),
        out_shape=jax.ShapeDtypeStruct((NHI, NLO), jnp.float32),
    )(ei)


BR = 1000


def _dinv_of(deg_ref):
    return lax.rsqrt(deg_ref[...] + 1.0)


def _mm_scale_body(x_ref, w_ref, deg_ref, o_ref):
    dinv = _dinv_of(deg_ref)
    xw = jnp.dot(x_ref[...], w_ref[...], preferred_element_type=jnp.float32)
    o_ref[...] = xw * dinv


def _mm_scale(x, W1, degc):
    return pl.pallas_call(
        _mm_scale_body,
        grid=(N // BR,),
        in_specs=[
            pl.BlockSpec((BR, DF), lambda i: (i, 0)),
            pl.BlockSpec((DF, DF), lambda i: (0, 0)),
            pl.BlockSpec((BR, 1), lambda i: (i, 0)),
        ],
        out_specs=pl.BlockSpec((BR, DF), lambda i: (i, 0)),
        out_shape=jax.ShapeDtypeStruct((N, DF), jnp.float32),
    )(x, W1, degc)


def _layer2_body(p_ref, z_ref, deg_ref, b1_ref, o_ref):
    dinv = _dinv_of(deg_ref)
    agg = (p_ref[0] + p_ref[1] - z_ref[...]) * dinv + b1_ref[...]
    o_ref[...] = jnp.maximum(agg, 0.0) * dinv


def _layer2(p1, z, degc, b1r):
    return pl.pallas_call(
        _layer2_body,
        grid=(N // BR,),
        in_specs=[
            pl.BlockSpec((NC, BR, DF), lambda i: (0, i, 0)),
            pl.BlockSpec((BR, DF), lambda i: (i, 0)),
            pl.BlockSpec((BR, 1), lambda i: (i, 0)),
            pl.BlockSpec((1, DF), lambda i: (0, 0)),
        ],
        out_specs=pl.BlockSpec((BR, DF), lambda i: (i, 0)),
        out_shape=jax.ShapeDtypeStruct((N, DF), jnp.float32),
    )(p1, z, degc, b1r)


def _final_body(p_ref, z2_ref, deg_ref, w2_ref, b2_ref, o_ref):
    dinv = _dinv_of(deg_ref)
    agg = (p_ref[0] + p_ref[1] - z2_ref[...]) * dinv
    v = jnp.dot(agg, w2_ref[...], preferred_element_type=jnp.float32)
    v = v + b2_ref[...]
    col = lax.broadcasted_iota(jnp.int32, (BR, DC), 1)
    valid = col < 40
    vm = jnp.where(valid, v, jnp.float32(-1e30))
    m = jnp.max(vm, axis=1, keepdims=True)
    ex = jnp.where(valid, jnp.exp(v - m), 0.0)
    lse = jnp.log(jnp.sum(ex, axis=1, keepdims=True))
    o_ref[...] = v - m - lse


def _final(p2, z2, degc, W2p, b2r):
    return pl.pallas_call(
        _final_body,
        grid=(N // BR,),
        in_specs=[
            pl.BlockSpec((NC, BR, DF), lambda i: (0, i, 0)),
            pl.BlockSpec((BR, DF), lambda i: (i, 0)),
            pl.BlockSpec((BR, 1), lambda i: (i, 0)),
            pl.BlockSpec((DF, DC), lambda i: (0, 0)),
            pl.BlockSpec((1, DC), lambda i: (0, 0)),
        ],
        out_specs=pl.BlockSpec((BR, DC), lambda i: (i, 0)),
        out_shape=jax.ShapeDtypeStruct((N, DC), jnp.float32),
    )(p2, z2, degc, W2p, b2r)


def kernel(x, edge_index, W1, b1, W2, b2):
    ei = edge_index.astype(jnp.int32)
    src = ei[0]
    dst = ei[1]
    deg2d = _hist(ei)
    degc = deg2d.reshape(NHI * NLO)[:N].reshape(N, 1)
    z = _mm_scale(x, W1, degc)
    p1 = _agg128(z, src, dst).reshape(NC, N, DF)
    z2 = _layer2(p1, z, degc, b1.reshape(1, DF))
    p2 = _agg128(z2, src, dst).reshape(NC, N, DF)
    W2p = jnp.pad(W2, ((0, 0), (0, DC - W2.shape[1])))
    b2r = jnp.pad(b2, (0, DC - b2.shape[0])).reshape(1, DC)
    out64 = _final(p2, z2, degc, W2p, b2r)
    return out64[:, :40]


# speedup vs baseline: 30.4340x; 1.1401x over previous
import functools

import jax
import jax.numpy as jnp
from jax import lax
from jax.experimental import pallas as pl
from jax.experimental.pallas import tpu as pltpu
from jax.experimental.pallas import tpu_sc as plsc

N = 10000
E = 320000
DF = 128
DC = 64

NC = 2
NS = 16
NW = NC * NS
EPT = E // NW
CH = 80
NCH = EPT // CH
U = 4
NWIN = NCH // U
TAIL = NCH - NWIN * U
RSTRIDE = 624
JR = 40
NJ = 16

_MESH = dict(core_axis_name="c", subcore_axis_name="s")


def _make_agg(D):
    scratch = (
        [pltpu.VMEM((CH,), jnp.int32) for _ in range(U)]
        + [pltpu.VMEM((CH,), jnp.int32) for _ in range(U)]
        + [pltpu.VMEM((CH, D), jnp.float32) for _ in range(U)]
        + [pltpu.VMEM_SHARED((N, D), jnp.float32)]
        + [pltpu.SemaphoreType.DMA for _ in range(4 * U)]
    )

    @functools.partial(
        pl.kernel,
        mesh=plsc.VectorSubcoreMesh(**_MESH),
        out_type=jax.ShapeDtypeStruct((NC * N, D), jnp.float32),
        scratch_types=scratch,
    )
    def agg(z_hbm, src_hbm, dst_hbm, out_hbm, *sc):
        src_v = sc[0:U]
        dst_v = sc[U:2 * U]
        rows_v = sc[2 * U:3 * U]
        acc = sc[3 * U]
        sems = sc[3 * U + 1:]
        qsem = sems[0:U]
        dsem = sems[U:2 * U]
        gsem = sems[2 * U:3 * U]
        ssem = sems[3 * U:4 * U]
        c = lax.axis_index("c")
        s = lax.axis_index("s")
        wid = s * NC + c
        row0 = s * RSTRIDE
        pltpu.sync_copy(z_hbm.at[pl.ds(row0, NJ * JR)],
                        acc.at[pl.ds(row0, NJ * JR)])
        base = wid * EPT
        for b in range(U):
            off = base + b * CH
            pltpu.async_copy(src_hbm.at[pl.ds(off, CH)], src_v[b], qsem[b])
            pltpu.async_copy(dst_hbm.at[pl.ds(off, CH)], dst_v[b], dsem[b])
        plsc.subcore_barrier()

        def window(w, carry):
            woff = base + w * U * CH
            hg = []
            for b in range(U):
                @pl.when(w > 0)
                def _drain(b=b):
                    pltpu.make_async_copy(rows_v[b], acc.at[dst_v[b]],
                                          ssem[b]).wait()
                    off = woff + b * CH
                    pltpu.async_copy(dst_hbm.at[pl.ds(off, CH)], dst_v[b],
                                     dsem[b])
                pltpu.make_async_copy(src_hbm.at[pl.ds(base, CH)],
                                      src_v[b], qsem[b]).wait()
                hg.append(pltpu.async_copy(z_hbm.at[src_v[b]], rows_v[b],
                                           gsem[b]))
            for b in range(U):
                hg[b].wait()
                pltpu.make_async_copy(dst_hbm.at[pl.ds(base, CH)],
                                      dst_v[b], dsem[b]).wait()
                pltpu.async_copy(rows_v[b], acc.at[dst_v[b]], ssem[b],
                                 add=True)

                @pl.when(w < NWIN - 1)
                def _next_src(b=b):
                    off = woff + U * CH + b * CH
                    pltpu.async_copy(src_hbm.at[pl.ds(off, CH)], src_v[b],
                                     qsem[b])
            return carry

        lax.fori_loop(0, NWIN, window, 0)
        for b in range(U):
            pltpu.make_async_copy(rows_v[b], acc.at[dst_v[b]],
                                  ssem[b]).wait()
        for t in range(TAIL):
            off = base + (NWIN * U + t) * CH
            pltpu.sync_copy(src_hbm.at[pl.ds(off, CH)], src_v[0])
            pltpu.sync_copy(dst_hbm.at[pl.ds(off, CH)], dst_v[0])
            pltpu.async_copy(z_hbm.at[src_v[0]], rows_v[0], gsem[0]).wait()
            pltpu.sync_copy(rows_v[0], acc.at[dst_v[0]], add=True)
        plsc.subcore_barrier()
        pltpu.sync_copy(acc.at[pl.ds(row0, NJ * JR)],
                        out_hbm.at[pl.ds(c * N + row0, NJ * JR)])

    return agg


_agg128 = _make_agg(DF)


EB = 16000
NHI = 80
NLO = 128


def _hist_body(ei_ref, o_ref):
    i = pl.program_id(0)

    @pl.when(i == 0)
    def _init():
        o_ref[...] = jnp.zeros_like(o_ref)

    drow = ei_ref[1:2, :]
    hi_t = (lax.broadcasted_iota(jnp.int32, (NHI, EB), 0)
            == (drow >> 7)).astype(jnp.float32)
    lo_t = (lax.broadcasted_iota(jnp.int32, (NLO, EB), 0)
            == (drow & 127)).astype(jnp.float32)
    o_ref[...] += lax.dot_general(hi_t, lo_t, (((1,), (1,)), ((), ())),
                                  preferred_element_type=jnp.float32)


def _hist(ei):
    return pl.pallas_call(
        _hist_body,
        grid=(E // EB,),
        in_specs=[pl.BlockSpec((2, EB), lambda i: (0, i))],
        out_specs=pl.BlockSpec((NHI, NLO), lambda i: (0, 0)),
        out_shape=jax.ShapeDtypeStruct((NHI, NLO), jnp.float32),
    )(ei)


BR = 1000


def _dinv_of(deg_ref):
    return lax.rsqrt(deg_ref[...] + 1.0)


def _mm_scale_body(x_ref, w_ref, deg_ref, o_ref):
    dinv = _dinv_of(deg_ref)
    xw = jnp.dot(x_ref[...], w_ref[...], preferred_element_type=jnp.float32)
    o_ref[...] = xw * dinv


def _mm_scale(x, W1, degc):
    return pl.pallas_call(
        _mm_scale_body,
        grid=(N // BR,),
        in_specs=[
            pl.BlockSpec((BR, DF), lambda i: (i, 0)),
            pl.BlockSpec((DF, DF), lambda i: (0, 0)),
            pl.BlockSpec((BR, 1), lambda i: (i, 0)),
        ],
        out_specs=pl.BlockSpec((BR, DF), lambda i: (i, 0)),
        out_shape=jax.ShapeDtypeStruct((N, DF), jnp.float32),
    )(x, W1, degc)


def _layer2_body(p_ref, z_ref, deg_ref, b1_ref, o_ref):
    dinv = _dinv_of(deg_ref)
    agg = (p_ref[0] + p_ref[1] - z_ref[...]) * dinv + b1_ref[...]
    o_ref[...] = jnp.maximum(agg, 0.0) * dinv


def _layer2(p1, z, degc, b1r):
    return pl.pallas_call(
        _layer2_body,
        grid=(N // BR,),
        in_specs=[
            pl.BlockSpec((NC, BR, DF), lambda i: (0, i, 0)),
            pl.BlockSpec((BR, DF), lambda i: (i, 0)),
            pl.BlockSpec((BR, 1), lambda i: (i, 0)),
            pl.BlockSpec((1, DF), lambda i: (0, 0)),
        ],
        out_specs=pl.BlockSpec((BR, DF), lambda i: (i, 0)),
        out_shape=jax.ShapeDtypeStruct((N, DF), jnp.float32),
    )(p1, z, degc, b1r)


def _final_body(p_ref, z2_ref, deg_ref, w2_ref, b2_ref, o_ref):
    dinv = _dinv_of(deg_ref)
    agg = (p_ref[0] + p_ref[1] - z2_ref[...]) * dinv
    v = jnp.dot(agg, w2_ref[...], preferred_element_type=jnp.float32)
    v = v + b2_ref[...]
    col = lax.broadcasted_iota(jnp.int32, (BR, DC), 1)
    valid = col < 40
    vm = jnp.where(valid, v, jnp.float32(-1e30))
    m = jnp.max(vm, axis=1, keepdims=True)
    ex = jnp.where(valid, jnp.exp(v - m), 0.0)
    lse = jnp.log(jnp.sum(ex, axis=1, keepdims=True))
    o_ref[...] = v - m - lse


def _final(p2, z2, degc, W2p, b2r):
    return pl.pallas_call(
        _final_body,
        grid=(N // BR,),
        in_specs=[
            pl.BlockSpec((NC, BR, DF), lambda i: (0, i, 0)),
            pl.BlockSpec((BR, DF), lambda i: (i, 0)),
            pl.BlockSpec((BR, 1), lambda i: (i, 0)),
            pl.BlockSpec((DF, DC), lambda i: (0, 0)),
            pl.BlockSpec((1, DC), lambda i: (0, 0)),
        ],
        out_specs=pl.BlockSpec((BR, DC), lambda i: (i, 0)),
        out_shape=jax.ShapeDtypeStruct((N, DC), jnp.float32),
    )(p2, z2, degc, W2p, b2r)


def kernel(x, edge_index, W1, b1, W2, b2):
    ei = edge_index.astype(jnp.int32)
    src = ei[0]
    dst = ei[1]
    deg2d = _hist(ei)
    degc = deg2d.reshape(NHI * NLO)[:N].reshape(N, 1)
    z = _mm_scale(x, W1, degc)
    p1 = _agg128(z, src, dst).reshape(NC, N, DF)
    z2 = _layer2(p1, z, degc, b1.reshape(1, DF))
    p2 = _agg128(z2, src, dst).reshape(NC, N, DF)
    W2p = jnp.pad(W2, ((0, 0), (0, DC - W2.shape[1])))
    b2r = jnp.pad(b2, (0, DC - b2.shape[0])).reshape(1, DC)
    out64 = _final(p2, z2, degc, W2p, b2r)
    return out64[:, :40]
